# baseline (speedup 1.0000x reference)
import functools

import jax
import jax.numpy as jnp
from jax import lax
from jax.experimental import pallas as pl
from jax.experimental.pallas import tpu as pltpu
from jax.experimental.pallas import tpu_sc as plsc

N = 1600000
D = 8
NUM_SEG = 50000
SEG_PAD = 50048


_G0 = 100
_G1 = 125
_BG = 10
_BL = _BG * _G1 * 128


_NB = _G0 // _BG


def _mse_body(pred_ref, tp_ref, rec_hbm, pk_ref, rbuf, rsem):
    i = pl.program_id(0)
    sl = _G1 * 128
    slot = i % 2

    def rcopy(j, s):
        return pltpu.make_async_copy(rec_hbm.at[pl.ds(j * _BL, _BL)],
                                     rbuf.at[s], rsem)

    @pl.when(i == 0)
    def _():
        rcopy(0, 0).start()
        rcopy(1, 1).start()

    @pl.when(jnp.logical_and(i >= 1, i + 1 < _NB))
    def _():
        rcopy(i + 1, (i + 1) % 2).start()

    rcopy(i, slot).wait()

    for u in range(_BG):
        d = pred_ref[:, u * sl:(u + 1) * sl] - tp_ref[:, u * sl:(u + 1) * sl]
        mse = jnp.sum(d * d, axis=0)
        rec = rbuf[slot, pl.ds(u * sl, sl)]
        w = (rec > 0).astype(jnp.float32)
        au = lax.bitcast_convert_type(mse * w, jnp.uint32)
        au = (au + jnp.uint32(0x8000)) & jnp.uint32(0xFFFF0000)
        bu = lax.bitcast_convert_type(w, jnp.uint32) >> jnp.uint32(16)
        packed = lax.bitcast_convert_type(au | bu, jnp.float32)
        pk_ref[u:u + 1, 0:_G1, :] = packed.reshape(1, _G1, 128)


def _mse_pairs(pred_t, tp_t, rec):
    return pl.pallas_call(
        _mse_body,
        grid=(_NB,),
        in_specs=[
            pl.BlockSpec((D, _BL), lambda i: (0, i)),
            pl.BlockSpec((D, _BL), lambda i: (0, i)),
            pl.BlockSpec(memory_space=pl.ANY),
        ],
        out_specs=pl.BlockSpec((_BG, 128, 128), lambda i: (i, 0, 0)),
        out_shape=jax.ShapeDtypeStruct((_G0, 128, 128), jnp.float32),
        scratch_shapes=[
            pltpu.VMEM((2, _BL), jnp.int32),
            pltpu.SemaphoreType.DMA,
        ],
    )(pred_t, tp_t, rec)


_NW = 32
_CHR = 32
_CH = _CHR * 128
_SLR = 16000


def _seg_body(pk_hbm, pid_hbm, zer_hbm, out_m, out_c,
              bins_m, bins_c, pkbuf, pbuf, sem0, sem1, zsem):
    wid = lax.axis_index("c") * 16 + lax.axis_index("s")
    wid3 = wid * 3
    sems = (sem0, sem1)

    zm = pltpu.make_async_copy(zer_hbm, bins_m, zsem)
    zc = pltpu.make_async_copy(zer_hbm, bins_c, zsem)
    zm.start()
    zc.start()

    def mk(c):
        b = c % 2
        if c < 12:
            s = wid3 + c // 4
            part = c % 4
            r0 = part * _CHR
            npid = _CH if part < 3 else _SLR - 3 * _CH
        else:
            s = 96 + (wid >> 2)
            part = wid & 3
            r0 = part * _CHR
            npid = _CH
        pid_off = s * _SLR + r0 * 128
        if c < 12:
            pidcp = pltpu.make_async_copy(
                pid_hbm.at[pl.ds(pid_off, npid)],
                pbuf.at[b, pl.ds(0, npid)], sems[b])
        else:
            pidcp = pltpu.make_async_copy(
                pid_hbm.at[pl.ds(pid_off, 3712)],
                pbuf.at[b, pl.ds(0, 3712)], sems[b])
        tailcp = None
        if c >= 12:
            safe_off = jnp.where(part < 3, pid_off + 3712, pid_off)
            tailcp = pltpu.make_async_copy(
                pid_hbm.at[pl.ds(safe_off, 384)],
                pbuf.at[b, pl.ds(3712, 384)], sems[b])
        pkcp = pltpu.make_async_copy(pk_hbm.at[s, pl.ds(r0, _CHR)],
                                     pkbuf.at[b], sems[b])
        if tailcp is None:
            return (pkcp, pidcp)
        return (pkcp, pidcp, tailcp)

    hi = jnp.uint32(0xFFFF0000)
    sh = jnp.uint32(16)

    def load_row(b, i):
        vals = []
        for u in range(8):
            pv = pkbuf[b, i, pl.ds(u * 16, 16)]
            uu = plsc.bitcast(pv, jnp.uint32)
            mv = plsc.bitcast(uu & hi, jnp.float32)
            wv = plsc.bitcast(uu << sh, jnp.float32)
            vals += [pbuf[b, pl.ds(i * 128 + u * 16, 16)], mv, wv]
        return tuple(vals)

    def scat(car):
        for u in range(8):
            pidv, mv, wv = car[3 * u:3 * u + 3]
            plsc.addupdate_scatter(bins_m, [pidv], mv)
            plsc.addupdate_scatter(bins_c, [pidv], wv)

    def proc(b, nrows):
        def body(i, car, b=b):
            scat(car)
            return load_row(b, i)

        car = load_row(b, 0)
        car = lax.fori_loop(1, nrows, body, car)
        scat(car)

    def startall(ds):
        for d in ds:
            d.start()

    pending = {0: mk(0), 1: mk(1)}
    startall(pending[0])
    startall(pending[1])
    zm.wait()
    zc.wait()
    for c in range(13):
        b = c % 2
        ds = pending.pop(c)
        if c < 12:
            for d in ds:
                d.wait()
            proc(b, _CHR if c % 4 < 3 else 29)
        else:
            @pl.when(wid < 16)
            def _(ds=ds, b=b):
                for d in ds:
                    d.wait()
                nrows = jnp.where((wid & 3) < 3, _CHR, 29)
                proc(b, nrows)
        if c + 2 < 13:
            nxt = mk(c + 2)
            pending[c + 2] = nxt
            if c + 2 == 12:
                @pl.when(wid < 16)
                def _(nxt=nxt):
                    startall(nxt)
            else:
                startall(nxt)

    pltpu.sync_copy(bins_m, out_m.at[wid])
    pltpu.sync_copy(bins_c, out_c.at[wid])


def _seg_partials(pk, pid, zer):
    mesh = plsc.VectorSubcoreMesh(core_axis_name="c", subcore_axis_name="s",
                                  num_cores=2, num_subcores=16)
    fn = pl.kernel(
        _seg_body,
        out_type=(
            jax.ShapeDtypeStruct((_NW, SEG_PAD), jnp.float32),
            jax.ShapeDtypeStruct((_NW, SEG_PAD), jnp.float32),
        ),
        mesh=mesh,
        scratch_types=[
            pltpu.VMEM((SEG_PAD,), jnp.float32),
            pltpu.VMEM((SEG_PAD,), jnp.float32),
            pltpu.VMEM((2, _CHR, 128), jnp.float32),
            pltpu.VMEM((2, _CH), jnp.int32),
            pltpu.SemaphoreType.DMA,
            pltpu.SemaphoreType.DMA,
            pltpu.SemaphoreType.DMA,
        ],
        compiler_params=pltpu.CompilerParams(needs_layout_passes=False),
    )
    return fn(pk, pid, zer)


def _final_body(pm_ref, pc_ref, out_ref):
    m = jnp.sum(pm_ref[...], axis=0, keepdims=True)
    c = jnp.sum(pc_ref[...], axis=0, keepdims=True)
    idx = lax.broadcasted_iota(jnp.int32, (1, SEG_PAD), 1)
    has = c > 0
    valid = has & (idx > 0)
    per = jnp.where(valid, m / jnp.where(has, c, 1.0), 0.0)
    loss = jnp.sum(per)
    kcount = jnp.sum(valid.astype(jnp.float32))
    out_ref[0, 0] = 100.0 * loss / kcount


def _final(pm, pc):
    return pl.pallas_call(
        _final_body,
        out_shape=jax.ShapeDtypeStruct((1, 1), jnp.float32),
        out_specs=pl.BlockSpec(memory_space=pltpu.SMEM),
    )(pm, pc)


def kernel(W, beta, H, pred, Y, particle_id, track_params, reconstructable):
    pred_t = pred.T
    tp_t = track_params.T
    rec = reconstructable.astype(jnp.int32)
    pk = _mse_pairs(pred_t, tp_t, rec)
    pid = particle_id.astype(jnp.int32)
    zer = jnp.zeros((SEG_PAD,), jnp.float32)
    pm, pc = _seg_partials(pk, pid, zer)
    return _final(pm, pc)[0, 0]


# speedup vs baseline: 1.0729x; 1.0729x over previous
import functools

import jax
import jax.numpy as jnp
from jax import lax
from jax.experimental import pallas as pl
from jax.experimental.pallas import tpu as pltpu
from jax.experimental.pallas import tpu_sc as plsc

N = 1600000
D = 8
NUM_SEG = 50000
SEG_PAD = 50048


_G0 = 100
_G1 = 125
_BG = 20
_BL = _BG * _G1 * 128


_NB = _G0 // _BG


def _mse_body(pred_ref, tp_ref, rec_hbm, pk_ref, rbuf, rsem):
    i = pl.program_id(0)
    sl = _G1 * 128
    slot = i % 2

    def rcopy(j, s):
        return pltpu.make_async_copy(rec_hbm.at[pl.ds(j * _BL, _BL)],
                                     rbuf.at[s], rsem)

    @pl.when(i == 0)
    def _():
        rcopy(0, 0).start()
        rcopy(1, 1).start()

    @pl.when(jnp.logical_and(i >= 1, i + 1 < _NB))
    def _():
        rcopy(i + 1, (i + 1) % 2).start()

    rcopy(i, slot).wait()

    for u in range(_BG):
        d = pred_ref[:, u * sl:(u + 1) * sl] - tp_ref[:, u * sl:(u + 1) * sl]
        mse = jnp.sum(d * d, axis=0)
        rec = rbuf[slot, pl.ds(u * sl, sl)]
        w = (rec > 0).astype(jnp.float32)
        au = lax.bitcast_convert_type(mse * w, jnp.uint32)
        au = (au + jnp.uint32(0x8000)) & jnp.uint32(0xFFFF0000)
        bu = lax.bitcast_convert_type(w, jnp.uint32) >> jnp.uint32(16)
        packed = lax.bitcast_convert_type(au | bu, jnp.float32)
        pk_ref[u:u + 1, 0:_G1, :] = packed.reshape(1, _G1, 128)


def _mse_pairs(pred_t, tp_t, rec):
    return pl.pallas_call(
        _mse_body,
        grid=(_NB,),
        in_specs=[
            pl.BlockSpec((D, _BL), lambda i: (0, i)),
            pl.BlockSpec((D, _BL), lambda i: (0, i)),
            pl.BlockSpec(memory_space=pl.ANY),
        ],
        out_specs=pl.BlockSpec((_BG, 128, 128), lambda i: (i, 0, 0)),
        out_shape=jax.ShapeDtypeStruct((_G0, 128, 128), jnp.float32),
        scratch_shapes=[
            pltpu.VMEM((2, _BL), jnp.int32),
            pltpu.SemaphoreType.DMA,
        ],
    )(pred_t, tp_t, rec)


_NW = 32
_CHR = 32
_CH = _CHR * 128
_SLR = 16000


def _seg_body(pk_hbm, pid_hbm, out_m, out_c,
              bins_m, bins_c, pkbuf, pbuf, sem0, sem1):
    wid = lax.axis_index("c") * 16 + lax.axis_index("s")
    wid3 = wid * 3
    sems = (sem0, sem1)

    zero16 = jnp.zeros((16,), jnp.float32)

    def zb(i, carry):
        for u in range(8):
            bins_m[pl.ds(i * 128 + u * 16, 16)] = zero16
            bins_c[pl.ds(i * 128 + u * 16, 16)] = zero16
        return carry

    lax.fori_loop(0, SEG_PAD // 128, zb, 0)

    def mk(c):
        b = c % 2
        if c < 12:
            s = wid3 + c // 4
            part = c % 4
            r0 = part * _CHR
            npid = _CH if part < 3 else _SLR - 3 * _CH
        else:
            s = 96 + (wid >> 2)
            part = wid & 3
            r0 = part * _CHR
            npid = _CH
        pid_off = s * _SLR + r0 * 128
        if c < 12:
            pidcp = pltpu.make_async_copy(
                pid_hbm.at[pl.ds(pid_off, npid)],
                pbuf.at[b, pl.ds(0, npid)], sems[b])
        else:
            pidcp = pltpu.make_async_copy(
                pid_hbm.at[pl.ds(pid_off, 3712)],
                pbuf.at[b, pl.ds(0, 3712)], sems[b])
        tailcp = None
        if c >= 12:
            safe_off = jnp.where(part < 3, pid_off + 3712, pid_off)
            tailcp = pltpu.make_async_copy(
                pid_hbm.at[pl.ds(safe_off, 384)],
                pbuf.at[b, pl.ds(3712, 384)], sems[b])
        pkcp = pltpu.make_async_copy(pk_hbm.at[s, pl.ds(r0, _CHR)],
                                     pkbuf.at[b], sems[b])
        if tailcp is None:
            return (pkcp, pidcp)
        return (pkcp, pidcp, tailcp)

    hi = jnp.uint32(0xFFFF0000)
    sh = jnp.uint32(16)

    def load_row(b, i):
        vals = []
        for u in range(8):
            pv = pkbuf[b, i, pl.ds(u * 16, 16)]
            uu = plsc.bitcast(pv, jnp.uint32)
            mv = plsc.bitcast(uu & hi, jnp.float32)
            wv = plsc.bitcast(uu << sh, jnp.float32)
            vals += [pbuf[b, pl.ds(i * 128 + u * 16, 16)], mv, wv]
        return tuple(vals)

    def scat(car):
        for u in range(8):
            pidv, mv, wv = car[3 * u:3 * u + 3]
            plsc.addupdate_scatter(bins_m, [pidv], mv)
            plsc.addupdate_scatter(bins_c, [pidv], wv)

    def proc(b, nrows):
        def body(i, car, b=b):
            scat(car)
            return load_row(b, i)

        car = load_row(b, 0)
        car = lax.fori_loop(1, nrows, body, car)
        scat(car)

    def startall(ds):
        for d in ds:
            d.start()

    pending = {0: mk(0), 1: mk(1)}
    startall(pending[0])
    startall(pending[1])
    for c in range(13):
        b = c % 2
        ds = pending.pop(c)
        if c < 12:
            for d in ds:
                d.wait()
            proc(b, _CHR if c % 4 < 3 else 29)
        else:
            @pl.when(wid < 16)
            def _(ds=ds, b=b):
                for d in ds:
                    d.wait()
                nrows = jnp.where((wid & 3) < 3, _CHR, 29)
                proc(b, nrows)
        if c + 2 < 13:
            nxt = mk(c + 2)
            pending[c + 2] = nxt
            if c + 2 == 12:
                @pl.when(wid < 16)
                def _(nxt=nxt):
                    startall(nxt)
            else:
                startall(nxt)

    pltpu.sync_copy(bins_m, out_m.at[wid])
    pltpu.sync_copy(bins_c, out_c.at[wid])


def _seg_partials(pk, pid):
    mesh = plsc.VectorSubcoreMesh(core_axis_name="c", subcore_axis_name="s",
                                  num_cores=2, num_subcores=16)
    fn = pl.kernel(
        _seg_body,
        out_type=(
            jax.ShapeDtypeStruct((_NW, SEG_PAD), jnp.float32),
            jax.ShapeDtypeStruct((_NW, SEG_PAD), jnp.float32),
        ),
        mesh=mesh,
        scratch_types=[
            pltpu.VMEM((SEG_PAD,), jnp.float32),
            pltpu.VMEM((SEG_PAD,), jnp.float32),
            pltpu.VMEM((2, _CHR, 128), jnp.float32),
            pltpu.VMEM((2, _CH), jnp.int32),
            pltpu.SemaphoreType.DMA,
            pltpu.SemaphoreType.DMA,
        ],
        compiler_params=pltpu.CompilerParams(needs_layout_passes=False),
    )
    return fn(pk, pid)


def _final_body(pm_ref, pc_ref, out_ref):
    m = jnp.sum(pm_ref[...], axis=0, keepdims=True)
    c = jnp.sum(pc_ref[...], axis=0, keepdims=True)
    idx = lax.broadcasted_iota(jnp.int32, (1, SEG_PAD), 1)
    has = c > 0
    valid = has & (idx > 0)
    per = jnp.where(valid, m / jnp.where(has, c, 1.0), 0.0)
    loss = jnp.sum(per)
    kcount = jnp.sum(valid.astype(jnp.float32))
    out_ref[0, 0] = 100.0 * loss / kcount


def _final(pm, pc):
    return pl.pallas_call(
        _final_body,
        out_shape=jax.ShapeDtypeStruct((1, 1), jnp.float32),
        out_specs=pl.BlockSpec(memory_space=pltpu.SMEM),
    )(pm, pc)


def kernel(W, beta, H, pred, Y, particle_id, track_params, reconstructable):
    pred_t = pred.T
    tp_t = track_params.T
    rec = reconstructable.astype(jnp.int32)
    pk = _mse_pairs(pred_t, tp_t, rec)
    pid = particle_id.astype(jnp.int32)
    pm, pc = _seg_partials(pk, pid)
    return _final(pm, pc)[0, 0]


# speedup vs baseline: 1.1175x; 1.0416x over previous
import jax
import jax.numpy as jnp
from jax import lax
from jax.experimental import pallas as pl
from jax.experimental.pallas import tpu as pltpu
from jax.experimental.pallas import tpu_sc as plsc

N = 1600000
D = 8
NUM_SEG = 50000
SEG_PAD = 50048


_G0 = 100
_G1 = 125
_BG = 10
_BL = _BG * _G1 * 128


_NB = _G0 // _BG


def _mse_body(pred_ref, tp_ref, rec_hbm, pk_ref, rbuf, rsem):
    i = pl.program_id(0)
    sl = _G1 * 128
    slot = i % 2

    def rcopy(j, s):
        return pltpu.make_async_copy(rec_hbm.at[pl.ds(j * _BL, _BL)],
                                     rbuf.at[s], rsem)

    @pl.when(i == 0)
    def _():
        rcopy(0, 0).start()
        rcopy(1, 1).start()

    @pl.when(jnp.logical_and(i >= 1, i + 1 < _NB))
    def _():
        rcopy(i + 1, (i + 1) % 2).start()

    rcopy(i, slot).wait()

    for u in range(_BG):
        d = pred_ref[:, u * sl:(u + 1) * sl] - tp_ref[:, u * sl:(u + 1) * sl]
        mse = jnp.sum(d * d, axis=0)
        rec = rbuf[slot, pl.ds(u * sl, sl)]
        w = (rec > 0).astype(jnp.float32)
        au = lax.bitcast_convert_type(mse * w, jnp.uint32)
        au = (au + jnp.uint32(0x8000)) & jnp.uint32(0xFFFF0000)
        bu = lax.bitcast_convert_type(w, jnp.uint32) >> jnp.uint32(16)
        packed = lax.bitcast_convert_type(au | bu, jnp.float32)
        pk_ref[u:u + 1, 0:_G1, :] = packed.reshape(1, _G1, 128)


def _mse_pairs(pred_t, tp_t, rec):
    return pl.pallas_call(
        _mse_body,
        grid=(_NB,),
        in_specs=[
            pl.BlockSpec((D, _BL), lambda i: (0, i)),
            pl.BlockSpec((D, _BL), lambda i: (0, i)),
            pl.BlockSpec(memory_space=pl.ANY),
        ],
        out_specs=pl.BlockSpec((_BG, 128, 128), lambda i: (i, 0, 0)),
        out_shape=jax.ShapeDtypeStruct((_G0, 128, 128), jnp.float32),
        scratch_shapes=[
            pltpu.VMEM((2, _BL), jnp.int32),
            pltpu.SemaphoreType.DMA,
        ],
    )(pred_t, tp_t, rec)


_NW = 32
_CHR = 32
_CH = _CHR * 128
_SLR = 16000


def _seg_body(pk_hbm, pid_hbm, out_m, out_c,
              bins_m, bins_c, pkbuf, pbuf, sem0, sem1):
    wid = lax.axis_index("c") * 16 + lax.axis_index("s")
    wid3 = wid * 3
    sems = (sem0, sem1)

    zero16 = jnp.zeros((16,), jnp.float32)

    def zb(i, carry):
        for u in range(8):
            bins_m[pl.ds(i * 128 + u * 16, 16)] = zero16
            bins_c[pl.ds(i * 128 + u * 16, 16)] = zero16
        return carry

    lax.fori_loop(0, SEG_PAD // 128, zb, 0)

    def mk(c):
        b = c % 2
        if c < 12:
            s = wid3 + c // 4
            part = c % 4
            r0 = part * _CHR
            npid = _CH if part < 3 else _SLR - 3 * _CH
        else:
            s = 96 + (wid >> 2)
            part = wid & 3
            r0 = part * _CHR
            npid = _CH
        pid_off = s * _SLR + r0 * 128
        if c < 12:
            pidcp = pltpu.make_async_copy(
                pid_hbm.at[pl.ds(pid_off, npid)],
                pbuf.at[b, pl.ds(0, npid)], sems[b])
        else:
            pidcp = pltpu.make_async_copy(
                pid_hbm.at[pl.ds(pid_off, 3712)],
                pbuf.at[b, pl.ds(0, 3712)], sems[b])
        tailcp = None
        if c >= 12:
            safe_off = jnp.where(part < 3, pid_off + 3712, pid_off)
            tailcp = pltpu.make_async_copy(
                pid_hbm.at[pl.ds(safe_off, 384)],
                pbuf.at[b, pl.ds(3712, 384)], sems[b])
        pkcp = pltpu.make_async_copy(pk_hbm.at[s, pl.ds(r0, _CHR)],
                                     pkbuf.at[b], sems[b])
        if tailcp is None:
            return (pkcp, pidcp)
        return (pkcp, pidcp, tailcp)

    hi = jnp.uint32(0xFFFF0000)
    sh = jnp.uint32(16)

    def load_row(b, i):
        vals = []
        for u in range(8):
            pv = pkbuf[b, i, pl.ds(u * 16, 16)]
            uu = plsc.bitcast(pv, jnp.uint32)
            mv = plsc.bitcast(uu & hi, jnp.float32)
            wv = plsc.bitcast(uu << sh, jnp.float32)
            vals += [pbuf[b, pl.ds(i * 128 + u * 16, 16)], mv, wv]
        return tuple(vals)

    def scat(car):
        for u in range(8):
            pidv, mv, wv = car[3 * u:3 * u + 3]
            plsc.addupdate_scatter(bins_m, [pidv], mv)
            plsc.addupdate_scatter(bins_c, [pidv], wv)

    def proc(b, nrows):
        def body(i, car, b=b):
            scat(car)
            return load_row(b, i)

        car = load_row(b, 0)
        car = lax.fori_loop(1, nrows, body, car)
        scat(car)

    def startall(ds):
        for d in ds:
            d.start()

    pending = {0: mk(0), 1: mk(1)}
    startall(pending[0])
    startall(pending[1])
    for c in range(13):
        b = c % 2
        ds = pending.pop(c)
        if c < 12:
            for d in ds:
                d.wait()
            proc(b, _CHR if c % 4 < 3 else 29)
        else:
            @pl.when(wid < 16)
            def _(ds=ds, b=b):
                for d in ds:
                    d.wait()
                nrows = jnp.where((wid & 3) < 3, _CHR, 29)
                proc(b, nrows)
        if c + 2 < 13:
            nxt = mk(c + 2)
            pending[c + 2] = nxt
            if c + 2 == 12:
                @pl.when(wid < 16)
                def _(nxt=nxt):
                    startall(nxt)
            else:
                startall(nxt)

    pltpu.sync_copy(bins_m, out_m.at[wid])
    pltpu.sync_copy(bins_c, out_c.at[wid])


def _seg_partials(pk, pid):
    mesh = plsc.VectorSubcoreMesh(core_axis_name="c", subcore_axis_name="s",
                                  num_cores=2, num_subcores=16)
    fn = pl.kernel(
        _seg_body,
        out_type=(
            jax.ShapeDtypeStruct((_NW, SEG_PAD), jnp.float32),
            jax.ShapeDtypeStruct((_NW, SEG_PAD), jnp.float32),
        ),
        mesh=mesh,
        scratch_types=[
            pltpu.VMEM((SEG_PAD,), jnp.float32),
            pltpu.VMEM((SEG_PAD,), jnp.float32),
            pltpu.VMEM((2, _CHR, 128), jnp.float32),
            pltpu.VMEM((2, _CH), jnp.int32),
            pltpu.SemaphoreType.DMA,
            pltpu.SemaphoreType.DMA,
        ],
        compiler_params=pltpu.CompilerParams(needs_layout_passes=False),
    )
    return fn(pk, pid)


def _final_body(pm_ref, pc_ref, out_ref):
    m = jnp.sum(pm_ref[...], axis=0, keepdims=True)
    c = jnp.sum(pc_ref[...], axis=0, keepdims=True)
    idx = lax.broadcasted_iota(jnp.int32, (1, SEG_PAD), 1)
    has = c > 0
    valid = has & (idx > 0)
    per = jnp.where(valid, m / jnp.where(has, c, 1.0), 0.0)
    loss = jnp.sum(per)
    kcount = jnp.sum(valid.astype(jnp.float32))
    out_ref[0, 0] = 100.0 * loss / kcount


def _final(pm, pc):
    return pl.pallas_call(
        _final_body,
        out_shape=jax.ShapeDtypeStruct((1, 1), jnp.float32),
        out_specs=pl.BlockSpec(memory_space=pltpu.SMEM),
    )(pm, pc)


def kernel(W, beta, H, pred, Y, particle_id, track_params, reconstructable):
    pred_t = pred.T
    tp_t = track_params.T
    rec = reconstructable.astype(jnp.int32)
    pk = _mse_pairs(pred_t, tp_t, rec)
    pid = particle_id.astype(jnp.int32)
    pm, pc = _seg_partials(pk, pid)
    return _final(pm, pc)[0, 0]
